# baseline (speedup 1.0000x reference)
import functools

import jax
import jax.numpy as jnp
from jax import lax
from jax.experimental import pallas as pl
from jax.experimental.pallas import tpu as pltpu
from jax.experimental.pallas import tpu_sc as plsc

NC = 2
NS = 16
CH = 96
ZR = 16


def _make_seg_sum(n_pad, n_feat, chunks_per_tile):
    rows_per_tile = n_pad // NS

    mesh = plsc.VectorSubcoreMesh(core_axis_name="c", subcore_axis_name="s")

    @functools.partial(
        pl.kernel,
        out_type=jax.ShapeDtypeStruct((NC, n_pad, n_feat), jnp.float32),
        mesh=mesh,
        scratch_types=[
            pltpu.VMEM_SHARED((n_pad, n_feat), jnp.float32),
            pltpu.VMEM((CH,), jnp.int32),
            pltpu.VMEM((CH,), jnp.int32),
            pltpu.VMEM((CH,), jnp.int32),
            pltpu.VMEM((CH,), jnp.int32),
            pltpu.VMEM((CH, n_feat), jnp.float32),
            pltpu.VMEM((CH, n_feat), jnp.float32),
            pltpu.SemaphoreType.DMA,
            pltpu.SemaphoreType.DMA,
            pltpu.SemaphoreType.DMA,
            pltpu.SemaphoreType.DMA,
        ],
    )
    def seg_sum(x_hbm, srcp_hbm, dstp_hbm, init_hbm, out_hbm, acc,
                srcv0, srcv1, dstv0, dstv1, rows0, rows1,
                sg0, sg1, si0, si1):
        c = lax.axis_index("c")
        s = lax.axis_index("s")
        w = s * NC + c
        r0 = pl.multiple_of(s * rows_per_tile, 8)
        ebase = pl.multiple_of(w * chunks_per_tile * CH, 8)

        pltpu.sync_copy(init_hbm.at[c, pl.ds(r0, rows_per_tile)],
                        acc.at[pl.ds(r0, rows_per_tile)])
        plsc.subcore_barrier()

        def fetch_idx(ck, sv, dv, sem):
            off = pl.multiple_of(ebase + ck * CH, 8)
            pltpu.async_copy(srcp_hbm.at[pl.ds(off, CH)], sv, sem)
            pltpu.async_copy(dstp_hbm.at[pl.ds(off, CH)], dv, sem)

        def iwait(sv, dv, sem):
            pltpu.make_async_copy(srcp_hbm.at[pl.ds(0, CH)], sv, sem).wait()
            pltpu.make_async_copy(dstp_hbm.at[pl.ds(0, CH)], dv, sem).wait()

        def gwait(rbuf, sem):
            pltpu.make_async_copy(x_hbm.at[srcv0], rbuf, sem).wait()

        fetch_idx(0, srcv0, dstv0, si0)
        iwait(srcv0, dstv0, si0)
        pltpu.async_copy(x_hbm.at[srcv0], rows0, sg0)
        fetch_idx(1, srcv1, dstv1, si1)
        n2 = chunks_per_tile // 2

        @pl.loop(0, n2)
        def _(j2):
            a = j2 * 2
            iwait(srcv1, dstv1, si1)
            pltpu.async_copy(x_hbm.at[srcv1], rows1, sg1)
            gwait(rows0, sg0)
            pltpu.sync_copy(rows0, acc.at[dstv0], add=True)

            @pl.when(j2 < n2 - 1)
            def _():
                fetch_idx(a + 2, srcv0, dstv0, si0)

            gwait(rows1, sg1)
            pltpu.sync_copy(rows1, acc.at[dstv1], add=True)

            @pl.when(j2 < n2 - 1)
            def _():
                iwait(srcv0, dstv0, si0)
                pltpu.async_copy(x_hbm.at[srcv0], rows0, sg0)
                fetch_idx(a + 3, srcv1, dstv1, si1)

        plsc.subcore_barrier()
        pltpu.sync_copy(acc.at[pl.ds(r0, rows_per_tile)],
                        out_hbm.at[c, pl.ds(r0, rows_per_tile)])

    return seg_sum


def _dense_body(agg_ref, x_ref, w1a_ref, w1b_ref, b1_ref, w2a_ref, w2b_ref,
                b2_ref, p_ref, q_ref):
    agg = agg_ref[0] + agg_ref[1]
    h = jnp.dot(agg, w1a_ref[...], preferred_element_type=jnp.float32)
    h = h + jnp.dot(x_ref[...], w1b_ref[...], preferred_element_type=jnp.float32)
    h = jnp.maximum(h + b1_ref[...], 0.0)
    p_ref[...] = jnp.dot(h, w2a_ref[...], preferred_element_type=jnp.float32)
    q_ref[...] = (jnp.dot(h, w2b_ref[...], preferred_element_type=jnp.float32)
                  + b2_ref[...])


def _make_combine(n_cls):
    def _combine_body(parts_ref, out_ref):
        out_ref[...] = (parts_ref[0, :, :n_cls] + parts_ref[1, :, :n_cls])
    return _combine_body


def kernel(x, edge_index, W1_rel, b1, W1_root, W2_rel, b2, W2_root):
    n_nodes, d_feat = x.shape
    n_edges = edge_index.shape[1]
    d_hid = W1_rel.shape[0]
    n_cls = W2_rel.shape[0]
    cls_pad = 128

    ei = edge_index.astype(jnp.int32)
    src, dst = ei[0], ei[1]

    workers = NC * NS
    ept = -(-n_edges // workers)
    cpt = -(-ept // CH)
    cpt += cpt % 2
    ept_p = cpt * CH
    ppt = ept_p - ept

    want_rows = max(NS * min(ppt, 32), 1)
    n_pad = ((n_nodes + want_rows + 127) // 128) * 128
    avail = n_pad - n_nodes
    pr = max(1, min(ppt, avail // NS))

    tail = workers * ept - n_edges
    src_a = jnp.concatenate(
        [src, jnp.zeros((tail,), jnp.int32)]).reshape(workers, ept)
    dst_tail = n_nodes + jnp.arange(tail, dtype=jnp.int32) % avail
    dst_a = jnp.concatenate([dst, dst_tail]).reshape(workers, ept)

    sub = jnp.arange(workers, dtype=jnp.int32) // NC
    pad_rows = (n_nodes + sub[:, None] * pr
                + jnp.arange(ppt, dtype=jnp.int32)[None, :] % pr)
    srcp = jnp.concatenate(
        [src_a, jnp.zeros((workers, ppt), jnp.int32)], axis=1).reshape(-1)
    dstp = jnp.concatenate([dst_a, pad_rows], axis=1).reshape(-1)

    seg1 = _make_seg_sum(n_pad, d_feat, cpt)
    init1 = jnp.zeros((NC, n_pad, d_feat), jnp.float32)
    agg1 = seg1(x, srcp, dstp, init1)

    w1a = W1_rel.T
    w1b = W1_root.T
    w2a = jnp.zeros((d_hid, cls_pad), jnp.float32).at[:, :n_cls].set(W2_rel.T)
    w2b = jnp.zeros((d_hid, cls_pad), jnp.float32).at[:, :n_cls].set(W2_root.T)
    b2p = jnp.zeros((1, cls_pad), jnp.float32).at[0, :n_cls].set(b2)

    tn = 1000
    grid = (n_nodes // tn,)
    p, q = pl.pallas_call(
        _dense_body,
        grid=grid,
        in_specs=[
            pl.BlockSpec((NC, tn, d_feat), lambda i: (0, i, 0)),
            pl.BlockSpec((tn, d_feat), lambda i: (i, 0)),
            pl.BlockSpec((d_feat, d_hid), lambda i: (0, 0)),
            pl.BlockSpec((d_feat, d_hid), lambda i: (0, 0)),
            pl.BlockSpec((1, d_hid), lambda i: (0, 0)),
            pl.BlockSpec((d_hid, cls_pad), lambda i: (0, 0)),
            pl.BlockSpec((d_hid, cls_pad), lambda i: (0, 0)),
            pl.BlockSpec((1, cls_pad), lambda i: (0, 0)),
        ],
        out_specs=[
            pl.BlockSpec((tn, cls_pad), lambda i: (i, 0)),
            pl.BlockSpec((tn, cls_pad), lambda i: (i, 0)),
        ],
        out_shape=[
            jax.ShapeDtypeStruct((n_nodes, cls_pad), jnp.float32),
            jax.ShapeDtypeStruct((n_nodes, cls_pad), jnp.float32),
        ],
    )(agg1, x, w1a, w1b, b1.reshape(1, -1), w2a, w2b, b2p)

    seg2 = _make_seg_sum(n_pad, cls_pad, cpt)
    init2 = jnp.zeros((NC, n_pad, cls_pad), jnp.float32).at[0, :n_nodes].set(q)
    agg2 = seg2(p, srcp, dstp, init2)

    out = pl.pallas_call(
        _make_combine(n_cls),
        grid=grid,
        in_specs=[pl.BlockSpec((NC, tn, cls_pad), lambda i: (0, i, 0))],
        out_specs=pl.BlockSpec((tn, n_cls), lambda i: (i, 0)),
        out_shape=jax.ShapeDtypeStruct((n_nodes, n_cls), jnp.float32),
    )(agg2)
    return out


# speedup vs baseline: 1.3974x; 1.3974x over previous
import functools

import jax
import jax.numpy as jnp
from jax import lax
from jax.experimental import pallas as pl
from jax.experimental.pallas import tpu as pltpu
from jax.experimental.pallas import tpu_sc as plsc

NC = 2
NS = 16
CH = 80
ZR = 16


def _make_seg_sum(n_pad, n_feat, chunks_per_tile):
    rows_per_tile = n_pad // NS

    mesh = plsc.VectorSubcoreMesh(core_axis_name="c", subcore_axis_name="s")

    @functools.partial(
        pl.kernel,
        out_type=jax.ShapeDtypeStruct((NC, n_pad, n_feat), jnp.float32),
        mesh=mesh,
        scratch_types=[
            pltpu.VMEM_SHARED((n_pad, n_feat), jnp.float32),
            pltpu.VMEM((CH,), jnp.int32),
            pltpu.VMEM((CH,), jnp.int32),
            pltpu.VMEM((CH,), jnp.int32),
            pltpu.VMEM((CH,), jnp.int32),
            pltpu.VMEM((CH, n_feat), jnp.float32),
            pltpu.VMEM((CH, n_feat), jnp.float32),
            pltpu.SemaphoreType.DMA,
            pltpu.SemaphoreType.DMA,
            pltpu.SemaphoreType.DMA,
            pltpu.SemaphoreType.DMA,
        ],
    )
    def seg_sum(x_hbm, srcp_hbm, dstp_hbm, init_hbm, out_hbm, acc,
                srcv0, srcv1, dstv0, dstv1, rows0, rows1,
                sg0, sg1, si0, si1):
        c = lax.axis_index("c")
        s = lax.axis_index("s")
        w = s * NC + c
        r0 = pl.multiple_of(s * rows_per_tile, 8)
        ebase = pl.multiple_of(w * chunks_per_tile * CH, 8)

        pltpu.sync_copy(init_hbm.at[c, pl.ds(r0, rows_per_tile)],
                        acc.at[pl.ds(r0, rows_per_tile)])
        plsc.subcore_barrier()

        def fetch_idx(ck, sv, dv, sem):
            off = pl.multiple_of(ebase + ck * CH, 8)
            pltpu.async_copy(srcp_hbm.at[pl.ds(off, CH)], sv, sem)
            pltpu.async_copy(dstp_hbm.at[pl.ds(off, CH)], dv, sem)

        def iwait(sv, dv, sem):
            pltpu.make_async_copy(srcp_hbm.at[pl.ds(0, CH)], sv, sem).wait()
            pltpu.make_async_copy(dstp_hbm.at[pl.ds(0, CH)], dv, sem).wait()

        def gwait(rbuf, sem):
            pltpu.make_async_copy(x_hbm.at[srcv0], rbuf, sem).wait()

        fetch_idx(0, srcv0, dstv0, si0)
        iwait(srcv0, dstv0, si0)
        pltpu.async_copy(x_hbm.at[srcv0], rows0, sg0)
        fetch_idx(1, srcv1, dstv1, si1)
        n2 = chunks_per_tile // 2

        @pl.loop(0, n2)
        def _(j2):
            a = j2 * 2
            iwait(srcv1, dstv1, si1)
            pltpu.async_copy(x_hbm.at[srcv1], rows1, sg1)
            gwait(rows0, sg0)
            pltpu.sync_copy(rows0, acc.at[dstv0], add=True)

            @pl.when(j2 < n2 - 1)
            def _():
                fetch_idx(a + 2, srcv0, dstv0, si0)

            gwait(rows1, sg1)
            pltpu.sync_copy(rows1, acc.at[dstv1], add=True)

            @pl.when(j2 < n2 - 1)
            def _():
                iwait(srcv0, dstv0, si0)
                pltpu.async_copy(x_hbm.at[srcv0], rows0, sg0)
                fetch_idx(a + 3, srcv1, dstv1, si1)

        plsc.subcore_barrier()
        pltpu.sync_copy(acc.at[pl.ds(r0, rows_per_tile)],
                        out_hbm.at[c, pl.ds(r0, rows_per_tile)])

    return seg_sum


def _dense_body(agg_ref, x_ref, w1a_ref, w1b_ref, b1_ref, w2a_ref, w2b_ref,
                b2_ref, p_ref, q_ref):
    agg = agg_ref[0] + agg_ref[1]
    h = jnp.dot(agg, w1a_ref[...], preferred_element_type=jnp.float32)
    h = h + jnp.dot(x_ref[...], w1b_ref[...], preferred_element_type=jnp.float32)
    h = jnp.maximum(h + b1_ref[...], 0.0)
    p_ref[...] = jnp.dot(h, w2a_ref[...], preferred_element_type=jnp.float32)
    q_ref[...] = (jnp.dot(h, w2b_ref[...], preferred_element_type=jnp.float32)
                  + b2_ref[...])


def _make_combine(n_cls):
    def _combine_body(parts_ref, out_ref):
        out_ref[...] = (parts_ref[0, :, :n_cls] + parts_ref[1, :, :n_cls])
    return _combine_body


def kernel(x, edge_index, W1_rel, b1, W1_root, W2_rel, b2, W2_root):
    n_nodes, d_feat = x.shape
    n_edges = edge_index.shape[1]
    d_hid = W1_rel.shape[0]
    n_cls = W2_rel.shape[0]
    cls_pad = 128

    ei = edge_index.astype(jnp.int32)
    src, dst = ei[0], ei[1]

    workers = NC * NS
    ept = -(-n_edges // workers)
    cpt = -(-ept // CH)
    cpt += cpt % 2
    ept_p = cpt * CH
    ppt = ept_p - ept

    want_rows = max(NS * min(ppt, 32), 1)
    n_pad = ((n_nodes + want_rows + 127) // 128) * 128
    avail = n_pad - n_nodes
    pr = max(1, min(ppt, avail // NS))

    tail = workers * ept - n_edges
    src_a = jnp.concatenate(
        [src, jnp.zeros((tail,), jnp.int32)]).reshape(workers, ept)
    dst_tail = n_nodes + jnp.arange(tail, dtype=jnp.int32) % avail
    dst_a = jnp.concatenate([dst, dst_tail]).reshape(workers, ept)

    sub = jnp.arange(workers, dtype=jnp.int32) // NC
    pad_rows = (n_nodes + sub[:, None] * pr
                + jnp.arange(ppt, dtype=jnp.int32)[None, :] % pr)
    srcp = jnp.concatenate(
        [src_a, jnp.zeros((workers, ppt), jnp.int32)], axis=1).reshape(-1)
    dstp = jnp.concatenate([dst_a, pad_rows], axis=1).reshape(-1)

    seg1 = _make_seg_sum(n_pad, d_feat, cpt)
    init1 = jnp.zeros((NC, n_pad, d_feat), jnp.float32)
    agg1 = seg1(x, srcp, dstp, init1)

    w1a = W1_rel.T
    w1b = W1_root.T
    w2a = jnp.zeros((d_hid, cls_pad), jnp.float32).at[:, :n_cls].set(W2_rel.T)
    w2b = jnp.zeros((d_hid, cls_pad), jnp.float32).at[:, :n_cls].set(W2_root.T)
    b2p = jnp.zeros((1, cls_pad), jnp.float32).at[0, :n_cls].set(b2)

    tn = 1000
    grid = (n_nodes // tn,)
    p, q = pl.pallas_call(
        _dense_body,
        grid=grid,
        in_specs=[
            pl.BlockSpec((NC, tn, d_feat), lambda i: (0, i, 0)),
            pl.BlockSpec((tn, d_feat), lambda i: (i, 0)),
            pl.BlockSpec((d_feat, d_hid), lambda i: (0, 0)),
            pl.BlockSpec((d_feat, d_hid), lambda i: (0, 0)),
            pl.BlockSpec((1, d_hid), lambda i: (0, 0)),
            pl.BlockSpec((d_hid, cls_pad), lambda i: (0, 0)),
            pl.BlockSpec((d_hid, cls_pad), lambda i: (0, 0)),
            pl.BlockSpec((1, cls_pad), lambda i: (0, 0)),
        ],
        out_specs=[
            pl.BlockSpec((tn, cls_pad), lambda i: (i, 0)),
            pl.BlockSpec((tn, cls_pad), lambda i: (i, 0)),
        ],
        out_shape=[
            jax.ShapeDtypeStruct((n_nodes, cls_pad), jnp.float32),
            jax.ShapeDtypeStruct((n_nodes, cls_pad), jnp.float32),
        ],
    )(agg1, x, w1a, w1b, b1.reshape(1, -1), w2a, w2b, b2p)

    seg2 = _make_seg_sum(n_pad, cls_pad, cpt)
    init2 = jnp.zeros((NC, n_pad, cls_pad), jnp.float32).at[0, :n_nodes].set(q)
    agg2 = seg2(p, srcp, dstp, init2)

    out = pl.pallas_call(
        _make_combine(n_cls),
        grid=grid,
        in_specs=[pl.BlockSpec((NC, tn, cls_pad), lambda i: (0, i, 0))],
        out_specs=pl.BlockSpec((tn, n_cls), lambda i: (i, 0)),
        out_shape=jax.ShapeDtypeStruct((n_nodes, n_cls), jnp.float32),
    )(agg2)
    return out


# speedup vs baseline: 1.5370x; 1.0999x over previous
import functools

import jax
import jax.numpy as jnp
from jax import lax
from jax.experimental import pallas as pl
from jax.experimental.pallas import tpu as pltpu
from jax.experimental.pallas import tpu_sc as plsc

NC = 2
NS = 16
CH = 40
ZR = 16


def _make_seg_sum(n_pad, n_feat, chunks_per_tile):
    rows_per_tile = n_pad // NS

    mesh = plsc.VectorSubcoreMesh(core_axis_name="c", subcore_axis_name="s")

    @functools.partial(
        pl.kernel,
        out_type=jax.ShapeDtypeStruct((NC, n_pad, n_feat), jnp.float32),
        mesh=mesh,
        scratch_types=[
            pltpu.VMEM_SHARED((n_pad, n_feat), jnp.float32),
            pltpu.VMEM((CH,), jnp.int32),
            pltpu.VMEM((CH,), jnp.int32),
            pltpu.VMEM((CH,), jnp.int32),
            pltpu.VMEM((CH,), jnp.int32),
            pltpu.VMEM((CH, n_feat), jnp.float32),
            pltpu.VMEM((CH, n_feat), jnp.float32),
            pltpu.SemaphoreType.DMA,
            pltpu.SemaphoreType.DMA,
            pltpu.SemaphoreType.DMA,
            pltpu.SemaphoreType.DMA,
        ],
    )
    def seg_sum(x_hbm, srcp_hbm, dstp_hbm, init_hbm, out_hbm, acc,
                srcv0, srcv1, dstv0, dstv1, rows0, rows1,
                sg0, sg1, si0, si1):
        c = lax.axis_index("c")
        s = lax.axis_index("s")
        w = s * NC + c
        r0 = pl.multiple_of(s * rows_per_tile, 8)
        ebase = pl.multiple_of(w * chunks_per_tile * CH, 8)

        pltpu.sync_copy(init_hbm.at[c, pl.ds(r0, rows_per_tile)],
                        acc.at[pl.ds(r0, rows_per_tile)])
        plsc.subcore_barrier()

        def fetch_idx(ck, sv, dv, sem):
            off = pl.multiple_of(ebase + ck * CH, 8)
            pltpu.async_copy(srcp_hbm.at[pl.ds(off, CH)], sv, sem)
            pltpu.async_copy(dstp_hbm.at[pl.ds(off, CH)], dv, sem)

        def iwait(sv, dv, sem):
            pltpu.make_async_copy(srcp_hbm.at[pl.ds(0, CH)], sv, sem).wait()
            pltpu.make_async_copy(dstp_hbm.at[pl.ds(0, CH)], dv, sem).wait()

        def gwait(rbuf, sem):
            pltpu.make_async_copy(x_hbm.at[srcv0], rbuf, sem).wait()

        fetch_idx(0, srcv0, dstv0, si0)
        iwait(srcv0, dstv0, si0)
        pltpu.async_copy(x_hbm.at[srcv0], rows0, sg0)
        fetch_idx(1, srcv1, dstv1, si1)
        n2 = chunks_per_tile // 2

        @pl.loop(0, n2)
        def _(j2):
            a = j2 * 2
            iwait(srcv1, dstv1, si1)
            pltpu.async_copy(x_hbm.at[srcv1], rows1, sg1)
            gwait(rows0, sg0)
            pltpu.sync_copy(rows0, acc.at[dstv0], add=True)

            @pl.when(j2 < n2 - 1)
            def _():
                fetch_idx(a + 2, srcv0, dstv0, si0)

            gwait(rows1, sg1)
            pltpu.sync_copy(rows1, acc.at[dstv1], add=True)

            @pl.when(j2 < n2 - 1)
            def _():
                iwait(srcv0, dstv0, si0)
                pltpu.async_copy(x_hbm.at[srcv0], rows0, sg0)
                fetch_idx(a + 3, srcv1, dstv1, si1)

        plsc.subcore_barrier()
        pltpu.sync_copy(acc.at[pl.ds(r0, rows_per_tile)],
                        out_hbm.at[c, pl.ds(r0, rows_per_tile)])

    return seg_sum


def _dense_body(agg_ref, x_ref, w1a_ref, w1b_ref, b1_ref, w2a_ref, w2b_ref,
                b2_ref, p_ref, q_ref):
    agg = agg_ref[0] + agg_ref[1]
    h = jnp.dot(agg, w1a_ref[...], preferred_element_type=jnp.float32)
    h = h + jnp.dot(x_ref[...], w1b_ref[...], preferred_element_type=jnp.float32)
    h = jnp.maximum(h + b1_ref[...], 0.0)
    p_ref[...] = jnp.dot(h, w2a_ref[...], preferred_element_type=jnp.float32)
    q_ref[...] = (jnp.dot(h, w2b_ref[...], preferred_element_type=jnp.float32)
                  + b2_ref[...])


def _make_combine(n_cls):
    def _combine_body(parts_ref, out_ref):
        out_ref[...] = (parts_ref[0, :, :n_cls] + parts_ref[1, :, :n_cls])
    return _combine_body


def kernel(x, edge_index, W1_rel, b1, W1_root, W2_rel, b2, W2_root):
    n_nodes, d_feat = x.shape
    n_edges = edge_index.shape[1]
    d_hid = W1_rel.shape[0]
    n_cls = W2_rel.shape[0]
    cls_pad = 128

    ei = edge_index.astype(jnp.int32)
    src, dst = ei[0], ei[1]

    workers = NC * NS
    ept = -(-n_edges // workers)
    cpt = -(-ept // CH)
    cpt += cpt % 2
    ept_p = cpt * CH
    ppt = ept_p - ept

    want_rows = max(NS * min(ppt, 32), 1)
    n_pad = ((n_nodes + want_rows + 127) // 128) * 128
    avail = n_pad - n_nodes
    pr = max(1, min(ppt, avail // NS))

    tail = workers * ept - n_edges
    src_a = jnp.concatenate(
        [src, jnp.zeros((tail,), jnp.int32)]).reshape(workers, ept)
    dst_tail = n_nodes + jnp.arange(tail, dtype=jnp.int32) % avail
    dst_a = jnp.concatenate([dst, dst_tail]).reshape(workers, ept)

    sub = jnp.arange(workers, dtype=jnp.int32) // NC
    pad_rows = (n_nodes + sub[:, None] * pr
                + jnp.arange(ppt, dtype=jnp.int32)[None, :] % pr)
    srcp = jnp.concatenate(
        [src_a, jnp.zeros((workers, ppt), jnp.int32)], axis=1).reshape(-1)
    dstp = jnp.concatenate([dst_a, pad_rows], axis=1).reshape(-1)

    seg1 = _make_seg_sum(n_pad, d_feat, cpt)
    init1 = jnp.zeros((NC, n_pad, d_feat), jnp.float32)
    agg1 = seg1(x, srcp, dstp, init1)

    w1a = W1_rel.T
    w1b = W1_root.T
    w2a = jnp.zeros((d_hid, cls_pad), jnp.float32).at[:, :n_cls].set(W2_rel.T)
    w2b = jnp.zeros((d_hid, cls_pad), jnp.float32).at[:, :n_cls].set(W2_root.T)
    b2p = jnp.zeros((1, cls_pad), jnp.float32).at[0, :n_cls].set(b2)

    tn = 1000
    grid = (n_nodes // tn,)
    p, q = pl.pallas_call(
        _dense_body,
        grid=grid,
        in_specs=[
            pl.BlockSpec((NC, tn, d_feat), lambda i: (0, i, 0)),
            pl.BlockSpec((tn, d_feat), lambda i: (i, 0)),
            pl.BlockSpec((d_feat, d_hid), lambda i: (0, 0)),
            pl.BlockSpec((d_feat, d_hid), lambda i: (0, 0)),
            pl.BlockSpec((1, d_hid), lambda i: (0, 0)),
            pl.BlockSpec((d_hid, cls_pad), lambda i: (0, 0)),
            pl.BlockSpec((d_hid, cls_pad), lambda i: (0, 0)),
            pl.BlockSpec((1, cls_pad), lambda i: (0, 0)),
        ],
        out_specs=[
            pl.BlockSpec((tn, cls_pad), lambda i: (i, 0)),
            pl.BlockSpec((tn, cls_pad), lambda i: (i, 0)),
        ],
        out_shape=[
            jax.ShapeDtypeStruct((n_nodes, cls_pad), jnp.float32),
            jax.ShapeDtypeStruct((n_nodes, cls_pad), jnp.float32),
        ],
    )(agg1, x, w1a, w1b, b1.reshape(1, -1), w2a, w2b, b2p)

    seg2 = _make_seg_sum(n_pad, cls_pad, cpt)
    init2 = jnp.zeros((NC, n_pad, cls_pad), jnp.float32).at[0, :n_nodes].set(q)
    agg2 = seg2(p, srcp, dstp, init2)

    out = pl.pallas_call(
        _make_combine(n_cls),
        grid=grid,
        in_specs=[pl.BlockSpec((NC, tn, cls_pad), lambda i: (0, i, 0))],
        out_specs=pl.BlockSpec((tn, n_cls), lambda i: (i, 0)),
        out_shape=jax.ShapeDtypeStruct((n_nodes, n_cls), jnp.float32),
    )(agg2)
    return out


# speedup vs baseline: 2.1431x; 1.3943x over previous
import functools

import jax
import jax.numpy as jnp
from jax import lax
from jax.experimental import pallas as pl
from jax.experimental.pallas import tpu as pltpu
from jax.experimental.pallas import tpu_sc as plsc

NC = 2
NS = 16
CH = 88
ZR = 16


def _make_seg_sum(n_pad, n_feat, chunks_per_tile):
    rows_per_tile = n_pad // NS

    mesh = plsc.VectorSubcoreMesh(core_axis_name="c", subcore_axis_name="s")

    @functools.partial(
        pl.kernel,
        out_type=jax.ShapeDtypeStruct((NC, n_pad, n_feat), jnp.float32),
        mesh=mesh,
        scratch_types=[
            pltpu.VMEM_SHARED((n_pad, n_feat), jnp.float32),
            pltpu.VMEM((CH,), jnp.int32),
            pltpu.VMEM((CH,), jnp.int32),
            pltpu.VMEM((CH,), jnp.int32),
            pltpu.VMEM((CH,), jnp.int32),
            pltpu.VMEM((CH, n_feat), jnp.float32),
            pltpu.VMEM((CH, n_feat), jnp.float32),
            pltpu.SemaphoreType.DMA,
            pltpu.SemaphoreType.DMA,
            pltpu.SemaphoreType.DMA,
            pltpu.SemaphoreType.DMA,
        ],
    )
    def seg_sum(x_hbm, srcp_hbm, dstp_hbm, init_hbm, out_hbm, acc,
                srcv0, srcv1, dstv0, dstv1, rows0, rows1,
                sg0, sg1, si0, si1):
        c = lax.axis_index("c")
        s = lax.axis_index("s")
        w = s * NC + c
        r0 = pl.multiple_of(s * rows_per_tile, 8)
        ebase = pl.multiple_of(w * chunks_per_tile * CH, 8)

        pltpu.sync_copy(init_hbm.at[c, pl.ds(r0, rows_per_tile)],
                        acc.at[pl.ds(r0, rows_per_tile)])
        plsc.subcore_barrier()

        def fetch_idx(ck, sv, dv, sem):
            off = pl.multiple_of(ebase + ck * CH, 8)
            pltpu.async_copy(srcp_hbm.at[pl.ds(off, CH)], sv, sem)
            pltpu.async_copy(dstp_hbm.at[pl.ds(off, CH)], dv, sem)

        def iwait(sv, dv, sem):
            pltpu.make_async_copy(srcp_hbm.at[pl.ds(0, CH)], sv, sem).wait()
            pltpu.make_async_copy(dstp_hbm.at[pl.ds(0, CH)], dv, sem).wait()

        def gwait(rbuf, sem):
            pltpu.make_async_copy(x_hbm.at[srcv0], rbuf, sem).wait()

        fetch_idx(0, srcv0, dstv0, si0)
        iwait(srcv0, dstv0, si0)
        pltpu.async_copy(x_hbm.at[srcv0], rows0, sg0)
        fetch_idx(1, srcv1, dstv1, si1)
        n2 = chunks_per_tile // 2

        @pl.loop(0, n2)
        def _(j2):
            a = j2 * 2
            iwait(srcv1, dstv1, si1)
            pltpu.async_copy(x_hbm.at[srcv1], rows1, sg1)
            gwait(rows0, sg0)
            pltpu.sync_copy(rows0, acc.at[dstv0], add=True)

            @pl.when(j2 < n2 - 1)
            def _():
                fetch_idx(a + 2, srcv0, dstv0, si0)

            gwait(rows1, sg1)
            pltpu.sync_copy(rows1, acc.at[dstv1], add=True)

            @pl.when(j2 < n2 - 1)
            def _():
                iwait(srcv0, dstv0, si0)
                pltpu.async_copy(x_hbm.at[srcv0], rows0, sg0)
                fetch_idx(a + 3, srcv1, dstv1, si1)

        plsc.subcore_barrier()
        pltpu.sync_copy(acc.at[pl.ds(r0, rows_per_tile)],
                        out_hbm.at[c, pl.ds(r0, rows_per_tile)])

    return seg_sum


def _dense_body(agg_ref, x_ref, w1a_ref, w1b_ref, b1_ref, w2a_ref, w2b_ref,
                b2_ref, p_ref, q_ref):
    agg = agg_ref[0] + agg_ref[1]
    h = jnp.dot(agg, w1a_ref[...], preferred_element_type=jnp.float32)
    h = h + jnp.dot(x_ref[...], w1b_ref[...], preferred_element_type=jnp.float32)
    h = jnp.maximum(h + b1_ref[...], 0.0)
    p_ref[...] = jnp.dot(h, w2a_ref[...], preferred_element_type=jnp.float32)
    q_ref[...] = (jnp.dot(h, w2b_ref[...], preferred_element_type=jnp.float32)
                  + b2_ref[...])


def _make_combine(n_cls):
    def _combine_body(parts_ref, out_ref):
        out_ref[...] = (parts_ref[0, :, :n_cls] + parts_ref[1, :, :n_cls])
    return _combine_body


def kernel(x, edge_index, W1_rel, b1, W1_root, W2_rel, b2, W2_root):
    n_nodes, d_feat = x.shape
    n_edges = edge_index.shape[1]
    d_hid = W1_rel.shape[0]
    n_cls = W2_rel.shape[0]
    cls_pad = 128

    ei = edge_index.astype(jnp.int32)
    src, dst = ei[0], ei[1]

    workers = NC * NS
    ept = -(-n_edges // workers)
    cpt = -(-ept // CH)
    cpt += cpt % 2
    ept_p = cpt * CH
    ppt = ept_p - ept

    want_rows = max(NS * min(ppt, 32), 1)
    n_pad = ((n_nodes + want_rows + 127) // 128) * 128
    avail = n_pad - n_nodes
    pr = max(1, min(ppt, avail // NS))

    tail = workers * ept - n_edges
    src_a = jnp.concatenate(
        [src, jnp.zeros((tail,), jnp.int32)]).reshape(workers, ept)
    dst_tail = n_nodes + jnp.arange(tail, dtype=jnp.int32) % avail
    dst_a = jnp.concatenate([dst, dst_tail]).reshape(workers, ept)

    sub = jnp.arange(workers, dtype=jnp.int32) // NC
    pad_rows = (n_nodes + sub[:, None] * pr
                + jnp.arange(ppt, dtype=jnp.int32)[None, :] % pr)
    pad_srcs = ((jnp.arange(workers, dtype=jnp.int32)[:, None] * 601
                 + jnp.arange(ppt, dtype=jnp.int32)[None, :] * 7) % n_nodes)
    srcp = jnp.concatenate([src_a, pad_srcs], axis=1).reshape(-1)
    dstp = jnp.concatenate([dst_a, pad_rows], axis=1).reshape(-1)

    seg1 = _make_seg_sum(n_pad, d_feat, cpt)
    init1 = jnp.zeros((NC, n_pad, d_feat), jnp.float32)
    agg1 = seg1(x, srcp, dstp, init1)

    w1a = W1_rel.T
    w1b = W1_root.T
    w2a = jnp.zeros((d_hid, cls_pad), jnp.float32).at[:, :n_cls].set(W2_rel.T)
    w2b = jnp.zeros((d_hid, cls_pad), jnp.float32).at[:, :n_cls].set(W2_root.T)
    b2p = jnp.zeros((1, cls_pad), jnp.float32).at[0, :n_cls].set(b2)

    tn = 1000
    grid = (n_nodes // tn,)
    p, q = pl.pallas_call(
        _dense_body,
        grid=grid,
        in_specs=[
            pl.BlockSpec((NC, tn, d_feat), lambda i: (0, i, 0)),
            pl.BlockSpec((tn, d_feat), lambda i: (i, 0)),
            pl.BlockSpec((d_feat, d_hid), lambda i: (0, 0)),
            pl.BlockSpec((d_feat, d_hid), lambda i: (0, 0)),
            pl.BlockSpec((1, d_hid), lambda i: (0, 0)),
            pl.BlockSpec((d_hid, cls_pad), lambda i: (0, 0)),
            pl.BlockSpec((d_hid, cls_pad), lambda i: (0, 0)),
            pl.BlockSpec((1, cls_pad), lambda i: (0, 0)),
        ],
        out_specs=[
            pl.BlockSpec((tn, cls_pad), lambda i: (i, 0)),
            pl.BlockSpec((tn, cls_pad), lambda i: (i, 0)),
        ],
        out_shape=[
            jax.ShapeDtypeStruct((n_nodes, cls_pad), jnp.float32),
            jax.ShapeDtypeStruct((n_nodes, cls_pad), jnp.float32),
        ],
    )(agg1, x, w1a, w1b, b1.reshape(1, -1), w2a, w2b, b2p)

    seg2 = _make_seg_sum(n_pad, cls_pad, cpt)
    init2 = jnp.zeros((NC, n_pad, cls_pad), jnp.float32).at[0, :n_nodes].set(q)
    agg2 = seg2(p, srcp, dstp, init2)

    out = pl.pallas_call(
        _make_combine(n_cls),
        grid=grid,
        in_specs=[pl.BlockSpec((NC, tn, cls_pad), lambda i: (0, i, 0))],
        out_specs=pl.BlockSpec((tn, n_cls), lambda i: (i, 0)),
        out_shape=jax.ShapeDtypeStruct((n_nodes, n_cls), jnp.float32),
    )(agg2)
    return out
